# baseline (speedup 1.0000x reference)
import jax
import jax.numpy as jnp
from jax.experimental import pallas as pl
from jax.experimental.pallas import tpu as pltpu

_B, _T, _N, _F, _L = 32, 12, 207, 64, 2


def _gat_path(x, W, a):
    h = jnp.dot(x, W, preferred_element_type=jnp.float32)
    e2 = jnp.dot(h, a.T, preferred_element_type=jnp.float32)
    e = e2[:, 0][:, None] + e2[:, 1][None, :]
    e = jnp.where(e >= 0, e, 0.2 * e)
    e = e - jnp.max(e, axis=-1, keepdims=True)
    p = jnp.exp(e)
    attn = p * (1.0 / jnp.sum(p, axis=-1, keepdims=True))
    out = jnp.dot(attn, h, preferred_element_type=jnp.float32)
    return out, attn


def _garnn_kernel(x_ref, wi_ref, wh_ref, ai_ref, ah_ref,
                  out_ref, hid_ref, attn_i_ref, attn_h_ref, h_scr):
    t = pl.program_id(1)

    @pl.when(t == 0)
    def _():
        h_scr[...] = jnp.zeros_like(h_scr)

    x = x_ref[0, 0]
    for l in range(_L):
        h_l = h_scr[l]
        gi, attn_i = _gat_path(x, wi_ref[l], ai_ref[l])
        gh, attn_h = _gat_path(h_l, wh_ref[l], ah_ref[l])
        gi_r, gi_z, gi_n = jnp.split(gi, 3, axis=-1)
        gh_r, gh_z, gh_n = jnp.split(gh, 3, axis=-1)
        r = jax.nn.sigmoid(gi_r + gh_r)
        z = jax.nn.sigmoid(gi_z + gh_z)
        n = jnp.tanh(gi_n + r * gh_n)
        h_new = (1.0 - z) * n + z * h_l
        h_scr[l] = h_new
        hid_ref[0, l] = h_new
        attn_i_ref[0, l] = attn_i
        attn_h_ref[0, l] = attn_h
        x = h_new
    out_ref[0, 0] = x


def kernel(input, Wi, Wh, ai, ah):
    Bb, Tt, Nn, Ff = input.shape
    Ll = Wi.shape[0]

    grid = (Bb, Tt)
    out_shapes = (
        jax.ShapeDtypeStruct((Bb, Tt, Nn, Ff), jnp.float32),
        jax.ShapeDtypeStruct((Bb, Ll, Nn, Ff), jnp.float32),
        jax.ShapeDtypeStruct((Bb, Ll, Nn, Nn), jnp.float32),
        jax.ShapeDtypeStruct((Bb, Ll, Nn, Nn), jnp.float32),
    )
    in_specs = [
        pl.BlockSpec((1, 1, Nn, Ff), lambda b, t: (b, t, 0, 0)),
        pl.BlockSpec((Ll, Ff, 3 * Ff), lambda b, t: (0, 0, 0)),
        pl.BlockSpec((Ll, Ff, 3 * Ff), lambda b, t: (0, 0, 0)),
        pl.BlockSpec((Ll, 2, 3 * Ff), lambda b, t: (0, 0, 0)),
        pl.BlockSpec((Ll, 2, 3 * Ff), lambda b, t: (0, 0, 0)),
    ]
    out_specs = (
        pl.BlockSpec((1, 1, Nn, Ff), lambda b, t: (b, t, 0, 0)),
        pl.BlockSpec((1, Ll, Nn, Ff), lambda b, t: (b, 0, 0, 0)),
        pl.BlockSpec((1, Ll, Nn, Nn), lambda b, t: (b, 0, 0, 0)),
        pl.BlockSpec((1, Ll, Nn, Nn), lambda b, t: (b, 0, 0, 0)),
    )
    output, hidden, attn_i, attn_h = pl.pallas_call(
        _garnn_kernel,
        grid=grid,
        in_specs=in_specs,
        out_specs=out_specs,
        out_shape=out_shapes,
        scratch_shapes=[pltpu.VMEM((Ll, Nn, Ff), jnp.float32)],
        compiler_params=pltpu.CompilerParams(
            dimension_semantics=("arbitrary", "arbitrary"),
        ),
    )(input, Wi, Wh, ai, ah)
    return (output, hidden, attn_i, attn_h)


# speedup vs baseline: 1.6374x; 1.6374x over previous
import jax
import jax.numpy as jnp
from jax.experimental import pallas as pl
from jax.experimental.pallas import tpu as pltpu

_B, _T, _N, _F, _L = 32, 12, 207, 64, 2
_P = 208
_BB = 8
_M = _BB * _P


def _attend(h, e2):
    e = e2[:, 0][:, None] + e2[:, 1][None, :]
    e = jnp.where(e >= 0, e, 0.2 * e)
    e = e - jnp.max(e, axis=-1, keepdims=True)
    p = jnp.exp(e)
    attn = p * (1.0 / jnp.sum(p, axis=-1, keepdims=True))
    out = jnp.dot(attn, h, preferred_element_type=jnp.float32)
    return out, attn


def _garnn_kernel(x_ref, wi_ref, wh_ref, ai_ref, ah_ref,
                  out_ref, hid_ref, attn_i_ref, attn_h_ref, h_scr):
    t = pl.program_id(1)

    @pl.when(t == 0)
    def _():
        h_scr[...] = jnp.zeros_like(h_scr)

    last = t == _T - 1
    x2 = x_ref[0, 0]
    for l in range(_L):
        hfull = h_scr[l]
        gi_f = jnp.dot(x2, wi_ref[l], preferred_element_type=jnp.float32)
        gh_f = jnp.dot(hfull, wh_ref[l], preferred_element_type=jnp.float32)
        e2i = jnp.dot(gi_f, ai_ref[l].T, preferred_element_type=jnp.float32)
        e2h = jnp.dot(gh_f, ah_ref[l].T, preferred_element_type=jnp.float32)
        for i in range(_BB):
            lo = i * _P
            hi_b = gi_f[lo:lo + _N]
            hh_b = gh_f[lo:lo + _N]
            gi, attn_i = _attend(hi_b, e2i[lo:lo + _N])
            gh, attn_h = _attend(hh_b, e2h[lo:lo + _N])
            r = jax.nn.sigmoid(gi[:, :_F] + gh[:, :_F])
            z = jax.nn.sigmoid(gi[:, _F:2 * _F] + gh[:, _F:2 * _F])
            n = jnp.tanh(gi[:, 2 * _F:] + r * gh[:, 2 * _F:])
            h_prev = h_scr[l, lo:lo + _N]
            h_new = (1.0 - z) * n + z * h_prev
            h_scr[l, lo:lo + _N] = h_new
            if l == _L - 1:
                out_ref[i, 0] = h_new

            @pl.when(last)
            def _(i=i, l=l, h_new=h_new, attn_i=attn_i, attn_h=attn_h):
                hid_ref[i, l] = h_new
                attn_i_ref[i, l] = attn_i
                attn_h_ref[i, l] = attn_h
        x2 = h_scr[l][...]


def kernel(input, Wi, Wh, ai, ah):
    Bb, Tt, Nn, Ff = input.shape
    Ll = Wi.shape[0]
    nb = Bb // _BB

    xp = jnp.pad(input, ((0, 0), (0, 0), (0, _P - Nn), (0, 0)))
    xp = xp.reshape(nb, _BB, Tt, _P, Ff).transpose(0, 2, 1, 3, 4)
    xp = xp.reshape(nb, Tt, _M, Ff)

    grid = (nb, Tt)
    out_shapes = (
        jax.ShapeDtypeStruct((Bb, Tt, Nn, Ff), jnp.float32),
        jax.ShapeDtypeStruct((Bb, Ll, Nn, Ff), jnp.float32),
        jax.ShapeDtypeStruct((Bb, Ll, Nn, Nn), jnp.float32),
        jax.ShapeDtypeStruct((Bb, Ll, Nn, Nn), jnp.float32),
    )
    in_specs = [
        pl.BlockSpec((1, 1, _M, Ff), lambda b, t: (b, t, 0, 0)),
        pl.BlockSpec((Ll, Ff, 3 * Ff), lambda b, t: (0, 0, 0)),
        pl.BlockSpec((Ll, Ff, 3 * Ff), lambda b, t: (0, 0, 0)),
        pl.BlockSpec((Ll, 2, 3 * Ff), lambda b, t: (0, 0, 0)),
        pl.BlockSpec((Ll, 2, 3 * Ff), lambda b, t: (0, 0, 0)),
    ]
    out_specs = (
        pl.BlockSpec((_BB, 1, Nn, Ff), lambda b, t: (b, t, 0, 0)),
        pl.BlockSpec((_BB, Ll, Nn, Ff), lambda b, t: (b, 0, 0, 0)),
        pl.BlockSpec((_BB, Ll, Nn, Nn), lambda b, t: (b, 0, 0, 0)),
        pl.BlockSpec((_BB, Ll, Nn, Nn), lambda b, t: (b, 0, 0, 0)),
    )
    output, hidden, attn_i, attn_h = pl.pallas_call(
        _garnn_kernel,
        grid=grid,
        in_specs=in_specs,
        out_specs=out_specs,
        out_shape=out_shapes,
        scratch_shapes=[pltpu.VMEM((Ll, _M, Ff), jnp.float32)],
        compiler_params=pltpu.CompilerParams(
            dimension_semantics=("arbitrary", "arbitrary"),
        ),
    )(xp, Wi, Wh, ai, ah)
    return (output, hidden, attn_i, attn_h)
